# baseline (speedup 1.0000x reference)
import jax
import jax.numpy as jnp
from jax import lax
from jax.experimental import pallas as pl
from jax.experimental.pallas import tpu as pltpu
from jax.experimental.pallas import tpu_sc as plsc

B, S, H = 4, 8192, 1024
L = 16
NC, NS = 2, 16
NW = NC * NS
CT = H // 128
GP = 128 // L
CHR = 256
U = 4

S_SC = 2048
S_TC = S - S_SC
NCH = S_SC // CHR
CHT = 2048
NCHT = S_TC // CHT


def _sc_pool(hid_ref, om_ref, os_ref, ot_ref,
             buf0, buf1, macc, sacc, tacc, stm, sts, stt, sem0, sem1):
    wid = lax.axis_index("s") * NC + lax.axis_index("c")
    b = wid // CT
    col0 = pl.multiple_of((wid % CT) * 128, 128)
    row0 = b * S
    bufs = (buf0, buf1)
    sems = (sem0, sem1)

    minf = jnp.full((L,), -jnp.inf, jnp.float32)
    zero = jnp.zeros((L,), jnp.float32)

    def copy_chunk(cc, k):
        r = pl.multiple_of(row0 + cc * CHR, CHR)
        return pltpu.make_async_copy(
            hid_ref.at[pl.ds(r, CHR), pl.ds(col0, 128)], bufs[k], sems[k])

    for j in range(GP):
        macc[j] = minf
        sacc[j] = zero
        tacc[j] = zero

    copy_chunk(0, 0).start()

    def process(buf):
        for j in range(GP):
                cs = pl.ds(j * L, L)

                def p1(r, mm, buf=buf, cs=cs):
                    m0, m1 = mm
                    rb = r * U
                    for u in range(0, U, 2):
                        m0 = jnp.maximum(m0, buf[rb + u, cs])
                        m1 = jnp.maximum(m1, buf[rb + u + 1, cs])
                    return m0, m1

                m0, m1 = lax.fori_loop(0, CHR // U, p1, (minf, minf))
                mc = jnp.maximum(m0, m1)
                mp = macc[j]
                mN = jnp.maximum(mp, mc)
                alpha = jnp.exp(mp - mN)
                s0 = sacc[j] * alpha
                t0 = tacc[j] * alpha
                s1 = zero
                t1 = zero

                def p2(r, carry, buf=buf, cs=cs, mN=mN):
                    a0, b0, a1, b1 = carry
                    rb = r * U
                    for u in range(0, U, 2):
                        x0 = buf[rb + u, cs]
                        x1 = buf[rb + u + 1, cs]
                        e0 = jnp.exp(x0 - mN)
                        e1 = jnp.exp(x1 - mN)
                        a0 += e0
                        b0 += x0 * e0
                        a1 += e1
                        b1 += x1 * e1
                    return (a0, b0, a1, b1)

                s0, t0, s1, t1 = lax.fori_loop(
                    0, CHR // U, p2, (s0, t0, s1, t1))
                macc[j] = mN
                sacc[j] = s0 + s1
                tacc[j] = t0 + t1

    def super_body(i, carry):
        cc0 = i * 2
        for k in (0, 1):
            cc = cc0 + k
            nxt = cc + 1

            @pl.when(nxt < NCH)
            def _():
                copy_chunk(nxt, (k + 1) % 2).start()

            copy_chunk(cc, k).wait()
            process(bufs[k])
        return carry

    lax.fori_loop(0, NCH // 2, super_body, 0)

    if NCH % 2:
        copy_chunk(NCH - 1, (NCH - 1) % 2).wait()
        process(bufs[(NCH - 1) % 2])

    for j in range(GP):
        cs = pl.ds(j * L, L)
        stm[cs] = macc[j]
        sts[cs] = sacc[j]
        stt[cs] = tacc[j]
    dst = pl.ds(b * H + col0, 128)
    pltpu.sync_copy(stm, om_ref.at[dst])
    pltpu.sync_copy(sts, os_ref.at[dst])
    pltpu.sync_copy(stt, ot_ref.at[dst])


_sc_pool_call = pl.kernel(
    _sc_pool,
    out_type=(
        jax.ShapeDtypeStruct((B * H,), jnp.float32),
        jax.ShapeDtypeStruct((B * H,), jnp.float32),
        jax.ShapeDtypeStruct((B * H,), jnp.float32),
    ),
    mesh=plsc.VectorSubcoreMesh(core_axis_name="c", subcore_axis_name="s"),
    scratch_types=[
        pltpu.VMEM((CHR, 128), jnp.float32),
        pltpu.VMEM((CHR, 128), jnp.float32),
        pltpu.VMEM((GP, L), jnp.float32),
        pltpu.VMEM((GP, L), jnp.float32),
        pltpu.VMEM((GP, L), jnp.float32),
        pltpu.VMEM((128,), jnp.float32),
        pltpu.VMEM((128,), jnp.float32),
        pltpu.VMEM((128,), jnp.float32),
        pltpu.SemaphoreType.DMA,
        pltpu.SemaphoreType.DMA,
    ],
)


def _tc_pool_body(x_ref, om_ref, os_ref, ot_ref, macc, sacc, tacc):
    c = pl.program_id(1)

    @pl.when(c == 0)
    def _():
        macc[...] = jnp.full((1, H), -jnp.inf, jnp.float32)
        sacc[...] = jnp.zeros((1, H), jnp.float32)
        tacc[...] = jnp.zeros((1, H), jnp.float32)

    x = x_ref[0]
    mc = jnp.max(x, axis=0, keepdims=True)
    mp = macc[...]
    mN = jnp.maximum(mp, mc)
    alpha = jnp.exp(mp - mN)
    e = jnp.exp(x - mN)
    ones = jnp.ones((1, CHT), jnp.float32)
    se = lax.dot_general(ones, e, (((1,), (0,)), ((), ())),
                         preferred_element_type=jnp.float32)
    te = lax.dot_general(ones, x * e, (((1,), (0,)), ((), ())),
                         preferred_element_type=jnp.float32)
    sacc[...] = sacc[...] * alpha + se
    tacc[...] = tacc[...] * alpha + te
    macc[...] = mN

    @pl.when(c == NCHT - 1)
    def _():
        om_ref[0] = macc[...]
        os_ref[0] = sacc[...]
        ot_ref[0] = tacc[...]


_tc_pool = pl.pallas_call(
    _tc_pool_body,
    grid=(B, NCHT),
    in_specs=[pl.BlockSpec((1, CHT, H),
                           lambda b, c: (b, S_SC // CHT + c, 0))],
    out_specs=[pl.BlockSpec((1, 1, H), lambda b, c: (b, 0, 0))] * 3,
    out_shape=[jax.ShapeDtypeStruct((B, 1, H), jnp.float32)] * 3,
    scratch_shapes=[pltpu.VMEM((1, H), jnp.float32)] * 3,
)


def _tc_final_body(msc_ref, ssc_ref, tsc_ref, mtc_ref, stc_ref, ttc_ref,
                   w_ref, b_ref, o_ref):
    msc = msc_ref[...].reshape(B, H)
    mtc = mtc_ref[...].reshape(B, H)
    mM = jnp.maximum(msc, mtc)
    asc = jnp.exp(msc - mM)
    atc = jnp.exp(mtc - mM)
    s = ssc_ref[...].reshape(B, H) * asc + stc_ref[...].reshape(B, H) * atc
    t = tsc_ref[...].reshape(B, H) * asc + ttc_ref[...].reshape(B, H) * atc
    pooled = t / s
    acc = lax.dot_general(
        pooled, w_ref[...], (((1,), (1,)), ((), ())),
        preferred_element_type=jnp.float32,
    )
    o_ref[...] = jnp.tanh(acc + b_ref[...])


_tc_final = pl.pallas_call(
    _tc_final_body,
    out_shape=jax.ShapeDtypeStruct((B, H), jnp.float32),
)


@jax.jit
def kernel(hidden_states, W, b):
    hid2 = hidden_states.reshape(B * S, H)
    msc, ssc, tsc = _sc_pool_call(hid2)
    mtc, stc, ttc = _tc_pool(hidden_states)
    return _tc_final(msc, ssc, tsc, mtc, stc, ttc,
                     W.astype(jnp.float32), b.reshape(1, H))


# speedup vs baseline: 1.0373x; 1.0373x over previous
import jax
import jax.numpy as jnp
from jax import lax
from jax.experimental import pallas as pl
from jax.experimental.pallas import tpu as pltpu
from jax.experimental.pallas import tpu_sc as plsc

B, S, H = 4, 8192, 1024
L = 16
NC, NS = 2, 16
NW = NC * NS
CT = H // 128
GP = 128 // L
CHR = 256
U = 16

S_SC = 2048
S_TC = S - S_SC
NCH = S_SC // CHR
CHT = 2048
NCHT = S_TC // CHT


def _sc_pool(hid_ref, om_ref, os_ref, ot_ref,
             buf0, buf1, macc, sacc, tacc, stm, sts, stt, sem0, sem1):
    wid = lax.axis_index("s") * NC + lax.axis_index("c")
    b = wid // CT
    col0 = pl.multiple_of((wid % CT) * 128, 128)
    row0 = b * S
    bufs = (buf0, buf1)
    sems = (sem0, sem1)

    minf = jnp.full((L,), -jnp.inf, jnp.float32)
    zero = jnp.zeros((L,), jnp.float32)

    def copy_chunk(cc, k):
        r = pl.multiple_of(row0 + cc * CHR, CHR)
        return pltpu.make_async_copy(
            hid_ref.at[pl.ds(r, CHR), pl.ds(col0, 128)], bufs[k], sems[k])

    for j in range(GP):
        macc[j] = minf
        sacc[j] = zero
        tacc[j] = zero

    copy_chunk(0, 0).start()

    def process(buf):
        for j in range(GP):
                cs = pl.ds(j * L, L)

                def p1(r, mm, buf=buf, cs=cs):
                    m0, m1 = mm
                    rb = r * U
                    for u in range(0, U, 2):
                        m0 = jnp.maximum(m0, buf[rb + u, cs])
                        m1 = jnp.maximum(m1, buf[rb + u + 1, cs])
                    return m0, m1

                m0, m1 = lax.fori_loop(0, CHR // U, p1, (minf, minf))
                mc = jnp.maximum(m0, m1)
                mp = macc[j]
                mN = jnp.maximum(mp, mc)
                alpha = jnp.exp(mp - mN)
                s0 = sacc[j] * alpha
                t0 = tacc[j] * alpha
                s1 = zero
                t1 = zero

                def p2(r, carry, buf=buf, cs=cs, mN=mN):
                    a0, b0, a1, b1 = carry
                    rb = r * U
                    for u in range(0, U, 2):
                        x0 = buf[rb + u, cs]
                        x1 = buf[rb + u + 1, cs]
                        e0 = jnp.exp(x0 - mN)
                        e1 = jnp.exp(x1 - mN)
                        a0 += e0
                        b0 += x0 * e0
                        a1 += e1
                        b1 += x1 * e1
                    return (a0, b0, a1, b1)

                s0, t0, s1, t1 = lax.fori_loop(
                    0, CHR // U, p2, (s0, t0, s1, t1))
                macc[j] = mN
                sacc[j] = s0 + s1
                tacc[j] = t0 + t1

    def super_body(i, carry):
        cc0 = i * 2
        for k in (0, 1):
            cc = cc0 + k
            nxt = cc + 1

            @pl.when(nxt < NCH)
            def _():
                copy_chunk(nxt, (k + 1) % 2).start()

            copy_chunk(cc, k).wait()
            process(bufs[k])
        return carry

    lax.fori_loop(0, NCH // 2, super_body, 0)

    if NCH % 2:
        copy_chunk(NCH - 1, (NCH - 1) % 2).wait()
        process(bufs[(NCH - 1) % 2])

    for j in range(GP):
        cs = pl.ds(j * L, L)
        stm[cs] = macc[j]
        sts[cs] = sacc[j]
        stt[cs] = tacc[j]
    dst = pl.ds(b * H + col0, 128)
    pltpu.sync_copy(stm, om_ref.at[dst])
    pltpu.sync_copy(sts, os_ref.at[dst])
    pltpu.sync_copy(stt, ot_ref.at[dst])


_sc_pool_call = pl.kernel(
    _sc_pool,
    out_type=(
        jax.ShapeDtypeStruct((B * H,), jnp.float32),
        jax.ShapeDtypeStruct((B * H,), jnp.float32),
        jax.ShapeDtypeStruct((B * H,), jnp.float32),
    ),
    mesh=plsc.VectorSubcoreMesh(core_axis_name="c", subcore_axis_name="s"),
    scratch_types=[
        pltpu.VMEM((CHR, 128), jnp.float32),
        pltpu.VMEM((CHR, 128), jnp.float32),
        pltpu.VMEM((GP, L), jnp.float32),
        pltpu.VMEM((GP, L), jnp.float32),
        pltpu.VMEM((GP, L), jnp.float32),
        pltpu.VMEM((128,), jnp.float32),
        pltpu.VMEM((128,), jnp.float32),
        pltpu.VMEM((128,), jnp.float32),
        pltpu.SemaphoreType.DMA,
        pltpu.SemaphoreType.DMA,
    ],
)


def _tc_pool_body(x_ref, om_ref, os_ref, ot_ref, macc, sacc, tacc):
    c = pl.program_id(1)

    @pl.when(c == 0)
    def _():
        macc[...] = jnp.full((1, H), -jnp.inf, jnp.float32)
        sacc[...] = jnp.zeros((1, H), jnp.float32)
        tacc[...] = jnp.zeros((1, H), jnp.float32)

    x = x_ref[0]
    mc = jnp.max(x, axis=0, keepdims=True)
    mp = macc[...]
    mN = jnp.maximum(mp, mc)
    alpha = jnp.exp(mp - mN)
    e = jnp.exp(x - mN)
    ones = jnp.ones((1, CHT), jnp.float32)
    se = lax.dot_general(ones, e, (((1,), (0,)), ((), ())),
                         preferred_element_type=jnp.float32)
    te = lax.dot_general(ones, x * e, (((1,), (0,)), ((), ())),
                         preferred_element_type=jnp.float32)
    sacc[...] = sacc[...] * alpha + se
    tacc[...] = tacc[...] * alpha + te
    macc[...] = mN

    @pl.when(c == NCHT - 1)
    def _():
        om_ref[0] = macc[...]
        os_ref[0] = sacc[...]
        ot_ref[0] = tacc[...]


_tc_pool = pl.pallas_call(
    _tc_pool_body,
    grid=(B, NCHT),
    in_specs=[pl.BlockSpec((1, CHT, H),
                           lambda b, c: (b, S_SC // CHT + c, 0))],
    out_specs=[pl.BlockSpec((1, 1, H), lambda b, c: (b, 0, 0))] * 3,
    out_shape=[jax.ShapeDtypeStruct((B, 1, H), jnp.float32)] * 3,
    scratch_shapes=[pltpu.VMEM((1, H), jnp.float32)] * 3,
)


def _tc_final_body(msc_ref, ssc_ref, tsc_ref, mtc_ref, stc_ref, ttc_ref,
                   w_ref, b_ref, o_ref):
    msc = msc_ref[...].reshape(B, H)
    mtc = mtc_ref[...].reshape(B, H)
    mM = jnp.maximum(msc, mtc)
    asc = jnp.exp(msc - mM)
    atc = jnp.exp(mtc - mM)
    s = ssc_ref[...].reshape(B, H) * asc + stc_ref[...].reshape(B, H) * atc
    t = tsc_ref[...].reshape(B, H) * asc + ttc_ref[...].reshape(B, H) * atc
    pooled = t / s
    acc = lax.dot_general(
        pooled, w_ref[...], (((1,), (1,)), ((), ())),
        preferred_element_type=jnp.float32,
    )
    o_ref[...] = jnp.tanh(acc + b_ref[...])


_tc_final = pl.pallas_call(
    _tc_final_body,
    out_shape=jax.ShapeDtypeStruct((B, H), jnp.float32),
)


@jax.jit
def kernel(hidden_states, W, b):
    hid2 = hidden_states.reshape(B * S, H)
    msc, ssc, tsc = _sc_pool_call(hid2)
    mtc, stc, ttc = _tc_pool(hidden_states)
    return _tc_final(msc, ssc, tsc, mtc, stc, ttc,
                     W.astype(jnp.float32), b.reshape(1, H))


# speedup vs baseline: 1.0574x; 1.0194x over previous
import jax
import jax.numpy as jnp
from jax import lax
from jax.experimental import pallas as pl
from jax.experimental.pallas import tpu as pltpu
from jax.experimental.pallas import tpu_sc as plsc

B, S, H = 4, 8192, 1024
L = 16
NC, NS = 2, 16
NW = NC * NS
CT = H // 128
GP = 128 // L
CHR = 256
U = 8

S_SC = 2048
S_TC = S - S_SC
NCH = S_SC // CHR
CHT = 2048
NCHT = S_TC // CHT


def _sc_pool(hid_ref, om_ref, os_ref, ot_ref,
             buf2, macc, sacc, tacc, stm, sts, stt, sem0, sem1):
    wid = lax.axis_index("s") * NC + lax.axis_index("c")
    b = wid // CT
    col0 = pl.multiple_of((wid % CT) * 128, 128)
    row0 = b * S
    sems = (sem0, sem1)

    minf = jnp.full((L,), -jnp.inf, jnp.float32)
    zero = jnp.zeros((L,), jnp.float32)

    def copy_chunk(cc, k):
        r = pl.multiple_of(row0 + cc * CHR, CHR)
        return pltpu.make_async_copy(
            hid_ref.at[pl.ds(r, CHR), pl.ds(col0, 128)],
            buf2.at[k], sems[k])

    for j in range(GP):
        macc[j] = minf
        sacc[j] = zero
        tacc[j] = zero

    copy_chunk(0, 0).start()

    def body(cc, carry):
        nxt = cc + 1
        par = lax.rem(cc, 2)

        @pl.when(jnp.logical_and(nxt < NCH, par == 0))
        def _():
            copy_chunk(nxt, 1).start()

        @pl.when(jnp.logical_and(nxt < NCH, par == 1))
        def _():
            copy_chunk(nxt, 0).start()

        @pl.when(par == 0)
        def _():
            copy_chunk(cc, 0).wait()

        @pl.when(par == 1)
        def _():
            copy_chunk(cc, 1).wait()

        for j in range(GP):
            cs = pl.ds(j * L, L)

            def p1(r, mm, cs=cs, par=par):
                m0, m1 = mm
                rb = r * U
                for u in range(0, U, 2):
                    m0 = jnp.maximum(m0, buf2[par, rb + u, cs])
                    m1 = jnp.maximum(m1, buf2[par, rb + u + 1, cs])
                return m0, m1

            m0, m1 = lax.fori_loop(0, CHR // U, p1, (minf, minf))
            mc = jnp.maximum(m0, m1)
            mp = macc[j]
            mN = jnp.maximum(mp, mc)
            alpha = jnp.exp(mp - mN)
            s0 = sacc[j] * alpha
            t0 = tacc[j] * alpha
            s1 = zero
            t1 = zero

            def p2(r, carry, cs=cs, mN=mN, par=par):
                a0, b0, a1, b1 = carry
                rb = r * U
                for u in range(0, U, 2):
                    x0 = buf2[par, rb + u, cs]
                    x1 = buf2[par, rb + u + 1, cs]
                    e0 = jnp.exp(x0 - mN)
                    e1 = jnp.exp(x1 - mN)
                    a0 += e0
                    b0 += x0 * e0
                    a1 += e1
                    b1 += x1 * e1
                return (a0, b0, a1, b1)

            s0, t0, s1, t1 = lax.fori_loop(
                0, CHR // U, p2, (s0, t0, s1, t1))
            macc[j] = mN
            sacc[j] = s0 + s1
            tacc[j] = t0 + t1
        return carry

    lax.fori_loop(0, NCH, body, 0)

    for j in range(GP):
        cs = pl.ds(j * L, L)
        stm[cs] = macc[j]
        sts[cs] = sacc[j]
        stt[cs] = tacc[j]
    dst = pl.ds(b * H + col0, 128)
    pltpu.sync_copy(stm, om_ref.at[dst])
    pltpu.sync_copy(sts, os_ref.at[dst])
    pltpu.sync_copy(stt, ot_ref.at[dst])


_sc_pool_call = pl.kernel(
    _sc_pool,
    out_type=(
        jax.ShapeDtypeStruct((B * H,), jnp.float32),
        jax.ShapeDtypeStruct((B * H,), jnp.float32),
        jax.ShapeDtypeStruct((B * H,), jnp.float32),
    ),
    mesh=plsc.VectorSubcoreMesh(core_axis_name="c", subcore_axis_name="s"),
    scratch_types=[
        pltpu.VMEM((2, CHR, 128), jnp.float32),
        pltpu.VMEM((GP, L), jnp.float32),
        pltpu.VMEM((GP, L), jnp.float32),
        pltpu.VMEM((GP, L), jnp.float32),
        pltpu.VMEM((128,), jnp.float32),
        pltpu.VMEM((128,), jnp.float32),
        pltpu.VMEM((128,), jnp.float32),
        pltpu.SemaphoreType.DMA,
        pltpu.SemaphoreType.DMA,
    ],
)


def _tc_pool_body(x_ref, om_ref, os_ref, ot_ref, macc, sacc, tacc):
    c = pl.program_id(1)

    @pl.when(c == 0)
    def _():
        macc[...] = jnp.full((1, H), -jnp.inf, jnp.float32)
        sacc[...] = jnp.zeros((1, H), jnp.float32)
        tacc[...] = jnp.zeros((1, H), jnp.float32)

    x = x_ref[0]
    mc = jnp.max(x, axis=0, keepdims=True)
    mp = macc[...]
    mN = jnp.maximum(mp, mc)
    alpha = jnp.exp(mp - mN)
    e = jnp.exp(x - mN)
    ones = jnp.ones((1, CHT), jnp.float32)
    se = lax.dot_general(ones, e, (((1,), (0,)), ((), ())),
                         preferred_element_type=jnp.float32)
    te = lax.dot_general(ones, x * e, (((1,), (0,)), ((), ())),
                         preferred_element_type=jnp.float32)
    sacc[...] = sacc[...] * alpha + se
    tacc[...] = tacc[...] * alpha + te
    macc[...] = mN

    @pl.when(c == NCHT - 1)
    def _():
        om_ref[0] = macc[...]
        os_ref[0] = sacc[...]
        ot_ref[0] = tacc[...]


_tc_pool = pl.pallas_call(
    _tc_pool_body,
    grid=(B, NCHT),
    in_specs=[pl.BlockSpec((1, CHT, H),
                           lambda b, c: (b, S_SC // CHT + c, 0))],
    out_specs=[pl.BlockSpec((1, 1, H), lambda b, c: (b, 0, 0))] * 3,
    out_shape=[jax.ShapeDtypeStruct((B, 1, H), jnp.float32)] * 3,
    scratch_shapes=[pltpu.VMEM((1, H), jnp.float32)] * 3,
)


def _tc_final_body(msc_ref, ssc_ref, tsc_ref, mtc_ref, stc_ref, ttc_ref,
                   w_ref, b_ref, o_ref):
    msc = msc_ref[...].reshape(B, H)
    mtc = mtc_ref[...].reshape(B, H)
    mM = jnp.maximum(msc, mtc)
    asc = jnp.exp(msc - mM)
    atc = jnp.exp(mtc - mM)
    s = ssc_ref[...].reshape(B, H) * asc + stc_ref[...].reshape(B, H) * atc
    t = tsc_ref[...].reshape(B, H) * asc + ttc_ref[...].reshape(B, H) * atc
    pooled = t / s
    acc = lax.dot_general(
        pooled, w_ref[...], (((1,), (1,)), ((), ())),
        preferred_element_type=jnp.float32,
    )
    o_ref[...] = jnp.tanh(acc + b_ref[...])


_tc_final = pl.pallas_call(
    _tc_final_body,
    out_shape=jax.ShapeDtypeStruct((B, H), jnp.float32),
)


@jax.jit
def kernel(hidden_states, W, b):
    hid2 = hidden_states.reshape(B * S, H)
    msc, ssc, tsc = _sc_pool_call(hid2)
    mtc, stc, ttc = _tc_pool(hidden_states)
    return _tc_final(msc, ssc, tsc, mtc, stc, ttc,
                     W.astype(jnp.float32), b.reshape(1, H))
